# Initial kernel scaffold; baseline (speedup 1.0000x reference)
import jax
import jax.numpy as jnp
from jax.experimental import pallas as pl


def kernel(x, W_ih0, W_hh0, b_ih0, b_hh0, W_ih1, W_hh1, b_ih1, b_hh1, Wl1, bl1, Wr1, Wl2, bl2, Wr2, fcW1, fcb1, fcW2, fcb2):
    raise NotImplementedError("write your pallas kernel here")


# speedup vs baseline: 256.2790x; 256.2790x over previous
import jax
import jax.numpy as jnp
from jax.experimental import pallas as pl
from jax.experimental.pallas import tpu as pltpu

_B, _T, _IN, _H = 8, 512, 8, 128
_H1, _OUTF, _OUTS, _NFC = 100, 128, 1, 8
_G = 4 * _H
_CT = 64


def _elu(v):
    return jnp.where(v > 0, v, jnp.exp(v) - 1.0)


def _fused_kernel(xT_ref, A0_ref, b0_ref, Whh0T_ref, C1_ref, b1_ref,
                  Wl1T_ref, bl1_ref, Wr1T_ref, Wl2T_ref, bl2_ref, Wr2T_ref,
                  W1T_ref, b1f_ref, W2bd_ref, b2r_ref,
                  out_ref, pre0_ref, hs_ref):
    f32 = jnp.float32
    pre0_ref[:, :] = (
        jnp.dot(xT_ref[:, :], A0_ref[:, :], preferred_element_type=f32)
        + b0_ref[:, :])
    Whh0T = Whh0T_ref[:, :]
    C1 = C1_ref[:, :]
    b1v = b1_ref[:, :]

    def step(t, carry):
        h0, c0, h1, c1, acc = carry
        g0 = pre0_ref[pl.ds(t * _B, _B), :] + jnp.dot(
            h0, Whh0T, preferred_element_type=f32)
        i0 = jax.nn.sigmoid(g0[:, 0:_H])
        f0 = jax.nn.sigmoid(g0[:, _H:2 * _H])
        gg0 = jnp.tanh(g0[:, 2 * _H:3 * _H])
        o0 = jax.nn.sigmoid(g0[:, 3 * _H:4 * _H])
        c0 = f0 * c0 + i0 * gg0
        h0 = o0 * jnp.tanh(c0)
        g1 = b1v + jnp.dot(jnp.concatenate([h0, h1], axis=1), C1,
                           preferred_element_type=f32)
        i1 = jax.nn.sigmoid(g1[:, 0:_H])
        f1 = jax.nn.sigmoid(g1[:, _H:2 * _H])
        gg1 = jnp.tanh(g1[:, 2 * _H:3 * _H])
        o1 = jax.nn.sigmoid(g1[:, 3 * _H:4 * _H])
        c1 = f1 * c1 + i1 * gg1
        h1 = o1 * jnp.tanh(c1)
        hs_ref[pl.ds(t * _B, _B), :] = h1
        return (h0, c0, h1, c1, acc + h1)

    z = jnp.zeros((_B, _H), f32)
    _, _, _, _, acc = jax.lax.fori_loop(0, _T, step, (z, z, z, z, z))

    mean_feat = acc * (1.0 / _T)
    mm1 = jnp.dot(mean_feat, Wl1T_ref[:, :],
                  preferred_element_type=f32) + bl1_ref[:, :]
    Wr1T = Wr1T_ref[:, :]

    def chunk(c, sumf1):
        rc = jnp.dot(hs_ref[pl.ds(c * _CT * _B, _CT * _B), :], Wr1T,
                     preferred_element_type=f32)
        f1c = _elu(rc.reshape(_CT, _B, _H1) + mm1[None, :, :])
        return sumf1 + jnp.sum(f1c, axis=0)

    sumf1 = jax.lax.fori_loop(0, _T // _CT, chunk, jnp.zeros((_B, _H1), f32))
    meanf1 = sumf1 * (1.0 / _T)
    f1_last = _elu(jnp.dot(hs_ref[(_T - 1) * _B:_T * _B, :], Wr1T,
                           preferred_element_type=f32) + mm1)

    f2 = (jnp.dot(meanf1, Wl2T_ref[:, :], preferred_element_type=f32)
          + bl2_ref[:, :]
          + jnp.dot(f1_last, Wr2T_ref[:, :], preferred_element_type=f32))

    zfc = jnp.maximum(
        jnp.dot(f2, W1T_ref[:, :], preferred_element_type=f32)
        + b1f_ref[:, :], 0.0)
    out_ref[:, :] = jnp.dot(zfc, W2bd_ref[:, :],
                            preferred_element_type=f32) + b2r_ref[:, :]


def kernel(x, W_ih0, W_hh0, b_ih0, b_hh0, W_ih1, W_hh1, b_ih1, b_hh1,
           Wl1, bl1, Wr1, Wl2, bl2, Wr2, fcW1, fcb1, fcW2, fcb2):
    f32 = jnp.float32
    xT = jnp.transpose(x, (1, 0, 2)).reshape(_T * _B, _IN)
    A0 = W_ih0.T
    b0 = (b_ih0 + b_hh0).reshape(1, _G)
    Whh0T = W_hh0.T
    C1 = jnp.concatenate([W_ih1.T, W_hh1.T], axis=0)
    b1 = (b_ih1 + b_hh1).reshape(1, _G)
    Wl1T = Wl1.T
    bl1r = bl1.reshape(1, _H1)
    Wr1T = Wr1.T
    Wl2T = Wl2.T
    bl2r = bl2.reshape(1, _OUTF)
    Wr2T = Wr2.T
    W1T = fcW1.reshape(_NFC * 64, _OUTF).T
    b1f = fcb1.reshape(1, _NFC * 64)
    eye = jnp.eye(_NFC, dtype=f32)
    W2bd = (fcW2[:, 0, :, None] * eye[:, None, :]).reshape(_NFC * 64, _NFC)
    b2r = fcb2.reshape(1, _NFC)

    out = pl.pallas_call(
        _fused_kernel,
        out_shape=jax.ShapeDtypeStruct((_B, _NFC), f32),
        scratch_shapes=[
            pltpu.VMEM((_T * _B, _G), f32),
            pltpu.VMEM((_T * _B, _H), f32),
        ],
    )(xT, A0, b0, Whh0T, C1, b1, Wl1T, bl1r, Wr1T, Wl2T, bl2r, Wr2T,
      W1T, b1f, W2bd, b2r)
    return jnp.transpose(out)[:, :, None]


# speedup vs baseline: 334.3763x; 1.3047x over previous
import jax
import jax.numpy as jnp
from jax.experimental import pallas as pl
from jax.experimental.pallas import tpu as pltpu

_B, _T, _IN, _H = 8, 512, 8, 128
_H1, _OUTF, _OUTS, _NFC = 100, 128, 1, 8
_G = 4 * _H
_CT = 64


def _elu(v):
    return jnp.where(v > 0, v, jnp.exp(v) - 1.0)


def _gates(s, c, off):
    i = s[:, off:off + _H]
    f = s[:, off + _H:off + 2 * _H]
    g = 2.0 * s[:, off + 2 * _H:off + 3 * _H] - 1.0
    o = s[:, off + 3 * _H:off + 4 * _H]
    cn = f * c + i * g
    return o * jnp.tanh(cn), cn


def _fused_kernel(xT_ref, A0_ref, b0_ref, b1_ref, Wbig_ref,
                  Wl1T_ref, bl1_ref, Wr1T_ref, Wl2T_ref, bl2_ref, Wr2T_ref,
                  W1T_ref, b1f_ref, W2bd_ref, b2r_ref,
                  out_ref, pre_ref, hs_ref):
    f32 = jnp.float32
    bf = jnp.bfloat16
    pre_ref[:, 0:_G] = (
        jnp.dot(xT_ref[:, :], A0_ref[:, :], preferred_element_type=f32)
        + b0_ref[:, :])
    pre_ref[:, _G:2 * _G] = jnp.broadcast_to(b1_ref[:, :], (_T * _B, _G))
    Wbig = Wbig_ref[:, :]

    s0 = jax.nn.sigmoid(pre_ref[0:_B, 0:_G])
    z = jnp.zeros((_B, _H), f32)
    h0, c0 = _gates(s0, z, 0)

    def step(t, carry):
        h0, c0, h1, c1, acc = carry
        hcat = jnp.concatenate([h0, h1], axis=1).astype(bf)
        gall = jnp.dot(hcat, Wbig, preferred_element_type=f32)
        s = jax.nn.sigmoid(gall + pre_ref[pl.ds(t * _B, _B), :])
        h0n, c0n = _gates(s, c0, 0)
        h1n, c1n = _gates(s, c1, _G)
        hs_ref[pl.ds((t - 1) * _B, _B), :] = h1n
        return (h0n, c0n, h1n, c1n, acc + h1n)

    h0, c0, h1, c1, acc = jax.lax.fori_loop(
        1, _T, step, (h0, c0, z, z, z))

    hcat = jnp.concatenate([h0, h1], axis=1).astype(bf)
    gall = jnp.dot(hcat, Wbig, preferred_element_type=f32)
    s = jax.nn.sigmoid(gall[:, _G:2 * _G] + b1_ref[:, :])
    h1, c1 = _gates(s, c1, 0)
    hs_ref[(_T - 1) * _B:_T * _B, :] = h1
    acc = acc + h1

    mean_feat = acc * (1.0 / _T)
    mm1 = jnp.dot(mean_feat, Wl1T_ref[:, :],
                  preferred_element_type=f32) + bl1_ref[:, :]
    Wr1T = Wr1T_ref[:, :]

    def chunk(c, sumf1):
        rc = jnp.dot(hs_ref[pl.ds(c * _CT * _B, _CT * _B), :], Wr1T,
                     preferred_element_type=f32)
        f1c = _elu(rc.reshape(_CT, _B, _H1) + mm1[None, :, :])
        return sumf1 + jnp.sum(f1c, axis=0)

    sumf1 = jax.lax.fori_loop(0, _T // _CT, chunk, jnp.zeros((_B, _H1), f32))
    meanf1 = sumf1 * (1.0 / _T)
    f1_last = _elu(jnp.dot(hs_ref[(_T - 1) * _B:_T * _B, :], Wr1T,
                           preferred_element_type=f32) + mm1)

    f2 = (jnp.dot(meanf1, Wl2T_ref[:, :], preferred_element_type=f32)
          + bl2_ref[:, :]
          + jnp.dot(f1_last, Wr2T_ref[:, :], preferred_element_type=f32))

    zfc = jnp.maximum(
        jnp.dot(f2, W1T_ref[:, :], preferred_element_type=f32)
        + b1f_ref[:, :], 0.0)
    out_ref[:, :] = jnp.dot(zfc, W2bd_ref[:, :],
                            preferred_element_type=f32) + b2r_ref[:, :]


def kernel(x, W_ih0, W_hh0, b_ih0, b_hh0, W_ih1, W_hh1, b_ih1, b_hh1,
           Wl1, bl1, Wr1, Wl2, bl2, Wr2, fcW1, fcb1, fcW2, fcb2):
    f32 = jnp.float32
    bf = jnp.bfloat16
    xT = jnp.transpose(x, (1, 0, 2)).reshape(_T * _B, _IN)
    gscale1 = jnp.concatenate(
        [jnp.ones((2 * _H,), f32), jnp.full((_H,), 2.0, f32),
         jnp.ones((_H,), f32)])
    A0 = W_ih0.T * gscale1[None, :]
    b0 = ((b_ih0 + b_hh0) * gscale1).reshape(1, _G)
    b1 = ((b_ih1 + b_hh1) * gscale1).reshape(1, _G)
    gscale2 = jnp.concatenate([gscale1, gscale1])
    top = jnp.concatenate([W_hh0.T, W_ih1.T], axis=1)
    bot = jnp.concatenate(
        [jnp.zeros((_H, _G), f32), W_hh1.T], axis=1)
    Wbig = (jnp.concatenate([top, bot], axis=0)
            * gscale2[None, :]).astype(bf)
    Wl1T = Wl1.T
    bl1r = bl1.reshape(1, _H1)
    Wr1T = Wr1.T
    Wl2T = Wl2.T
    bl2r = bl2.reshape(1, _OUTF)
    Wr2T = Wr2.T
    W1T = fcW1.reshape(_NFC * 64, _OUTF).T
    b1f = fcb1.reshape(1, _NFC * 64)
    eye = jnp.eye(_NFC, dtype=f32)
    W2bd = (fcW2[:, 0, :, None] * eye[:, None, :]).reshape(_NFC * 64, _NFC)
    b2r = fcb2.reshape(1, _NFC)

    out = pl.pallas_call(
        _fused_kernel,
        out_shape=jax.ShapeDtypeStruct((_B, _NFC), f32),
        scratch_shapes=[
            pltpu.VMEM((_T * _B, 2 * _G), f32),
            pltpu.VMEM((_T * _B, _H), f32),
        ],
    )(xT, A0, b0, b1, Wbig, Wl1T, bl1r, Wr1T, Wl2T, bl2r, Wr2T,
      W1T, b1f, W2bd, b2r)
    return jnp.transpose(out)[:, :, None]


# speedup vs baseline: 371.0827x; 1.1098x over previous
import jax
import jax.numpy as jnp
from jax.experimental import pallas as pl
from jax.experimental.pallas import tpu as pltpu

_B, _T, _IN, _H = 8, 512, 8, 128
_H1, _OUTF, _OUTS, _NFC = 100, 128, 1, 8
_G = 4 * _H
_CT = 64


def _elu(v):
    return jnp.where(v > 0, v, jnp.exp(v) - 1.0)


def _gates(s, c, off):
    i = s[:, off:off + _H]
    f = s[:, off + _H:off + 2 * _H]
    g = 2.0 * s[:, off + 2 * _H:off + 3 * _H] - 1.0
    o = s[:, off + 3 * _H:off + 4 * _H]
    cn = f * c + i * g
    return o * jnp.tanh(cn), cn


def _fused_kernel(xT_ref, A0_ref, b0_ref, b1_ref, Whh0_ref, C1_ref,
                  Wl1T_ref, bl1_ref, Wr1T_ref, Wl2T_ref, bl2_ref, Wr2T_ref,
                  W1T_ref, b1f_ref, W2bd_ref, b2r_ref,
                  out_ref, pre_ref, hs_ref):
    f32 = jnp.float32
    bf = jnp.bfloat16
    pre_ref[:, :] = (
        jnp.dot(xT_ref[:, :], A0_ref[:, :], preferred_element_type=f32)
        + b0_ref[:, :])

    s0 = jax.nn.sigmoid(pre_ref[0:_B, :])
    z = jnp.zeros((_B, _H), f32)
    h0, c0 = _gates(s0, z, 0)
    b1v = b1_ref[:, :]

    def step(t, carry):
        h0, c0, h1, c1, acc = carry
        g0 = jnp.dot(h0.astype(bf), Whh0_ref[:, :],
                     preferred_element_type=f32)
        hcat = jnp.concatenate([h0, h1], axis=1).astype(bf)
        g1 = jnp.dot(hcat, C1_ref[:, :], preferred_element_type=f32)
        s0 = jax.nn.sigmoid(g0 + pre_ref[pl.ds(t * _B, _B), :])
        s1 = jax.nn.sigmoid(g1 + b1v)
        h0n, c0n = _gates(s0, c0, 0)
        h1n, c1n = _gates(s1, c1, 0)
        hs_ref[pl.ds((t - 1) * _B, _B), :] = h1n
        return (h0n, c0n, h1n, c1n, acc + h1n)

    h0, c0, h1, c1, acc = jax.lax.fori_loop(
        1, _T, step, (h0, c0, z, z, z))

    hcat = jnp.concatenate([h0, h1], axis=1).astype(bf)
    g1 = jnp.dot(hcat, C1_ref[:, :], preferred_element_type=f32)
    s = jax.nn.sigmoid(g1 + b1v)
    h1, c1 = _gates(s, c1, 0)
    hs_ref[(_T - 1) * _B:_T * _B, :] = h1
    acc = acc + h1

    mean_feat = acc * (1.0 / _T)
    mm1 = jnp.dot(mean_feat, Wl1T_ref[:, :],
                  preferred_element_type=f32) + bl1_ref[:, :]
    Wr1T = Wr1T_ref[:, :]

    def chunk(c, sumf1):
        rc = jnp.dot(hs_ref[pl.ds(c * _CT * _B, _CT * _B), :], Wr1T,
                     preferred_element_type=f32)
        f1c = _elu(rc.reshape(_CT, _B, _H1) + mm1[None, :, :])
        return sumf1 + jnp.sum(f1c, axis=0)

    sumf1 = jax.lax.fori_loop(0, _T // _CT, chunk, jnp.zeros((_B, _H1), f32))
    meanf1 = sumf1 * (1.0 / _T)
    f1_last = _elu(jnp.dot(hs_ref[(_T - 1) * _B:_T * _B, :], Wr1T,
                           preferred_element_type=f32) + mm1)

    f2 = (jnp.dot(meanf1, Wl2T_ref[:, :], preferred_element_type=f32)
          + bl2_ref[:, :]
          + jnp.dot(f1_last, Wr2T_ref[:, :], preferred_element_type=f32))

    zfc = jnp.maximum(
        jnp.dot(f2, W1T_ref[:, :], preferred_element_type=f32)
        + b1f_ref[:, :], 0.0)
    out_ref[:, :] = jnp.dot(zfc, W2bd_ref[:, :],
                            preferred_element_type=f32) + b2r_ref[:, :]


def kernel(x, W_ih0, W_hh0, b_ih0, b_hh0, W_ih1, W_hh1, b_ih1, b_hh1,
           Wl1, bl1, Wr1, Wl2, bl2, Wr2, fcW1, fcb1, fcW2, fcb2):
    f32 = jnp.float32
    bf = jnp.bfloat16
    xT = jnp.transpose(x, (1, 0, 2)).reshape(_T * _B, _IN)
    gscale1 = jnp.concatenate(
        [jnp.ones((2 * _H,), f32), jnp.full((_H,), 2.0, f32),
         jnp.ones((_H,), f32)])
    A0 = W_ih0.T * gscale1[None, :]
    b0 = ((b_ih0 + b_hh0) * gscale1).reshape(1, _G)
    b1 = ((b_ih1 + b_hh1) * gscale1).reshape(1, _G)
    Whh0s = (W_hh0.T * gscale1[None, :]).astype(bf)
    C1s = (jnp.concatenate([W_ih1.T, W_hh1.T], axis=0)
           * gscale1[None, :]).astype(bf)
    Wl1T = Wl1.T
    bl1r = bl1.reshape(1, _H1)
    Wr1T = Wr1.T
    Wl2T = Wl2.T
    bl2r = bl2.reshape(1, _OUTF)
    Wr2T = Wr2.T
    W1T = fcW1.reshape(_NFC * 64, _OUTF).T
    b1f = fcb1.reshape(1, _NFC * 64)
    eye = jnp.eye(_NFC, dtype=f32)
    W2bd = (fcW2[:, 0, :, None] * eye[:, None, :]).reshape(_NFC * 64, _NFC)
    b2r = fcb2.reshape(1, _NFC)

    out = pl.pallas_call(
        _fused_kernel,
        out_shape=jax.ShapeDtypeStruct((_B, _NFC), f32),
        scratch_shapes=[
            pltpu.VMEM((_T * _B, _G), f32),
            pltpu.VMEM((_T * _B, _H), f32),
        ],
    )(xT, A0, b0, b1, Whh0s, C1s, Wl1T, bl1r, Wr1T, Wl2T, bl2r, Wr2T,
      W1T, b1f, W2bd, b2r)
    return jnp.transpose(out)[:, :, None]


# speedup vs baseline: 416.6362x; 1.1228x over previous
import jax
import jax.numpy as jnp
from jax.experimental import pallas as pl
from jax.experimental.pallas import tpu as pltpu

_B, _T, _IN, _H = 8, 512, 8, 128
_H1, _OUTF, _OUTS, _NFC = 100, 128, 1, 8
_G = 4 * _H
_CT = 64


def _elu(v):
    return jnp.where(v > 0, v, jnp.exp(v) - 1.0)


def _gates(s, c, off):
    i = s[:, off:off + _H]
    f = s[:, off + _H:off + 2 * _H]
    g = 2.0 * s[:, off + 2 * _H:off + 3 * _H] - 1.0
    o = s[:, off + 3 * _H:off + 4 * _H]
    cn = f * c + i * g
    return o * jnp.tanh(cn), cn


def _fused_kernel(xT_ref, A0_ref, b0_ref, b1_ref, Whh0_ref, C1_ref,
                  Wl1T_ref, bl1_ref, Wr1T_ref, Wl2T_ref, bl2_ref, Wr2T_ref,
                  W1T_ref, b1f_ref, W2bd_ref, b2r_ref,
                  out_ref, pre_ref, hs_ref):
    f32 = jnp.float32
    bf = jnp.bfloat16
    pre_ref[:, :] = (
        jnp.dot(xT_ref[:, :], A0_ref[:, :], preferred_element_type=f32)
        + b0_ref[:, :])

    s0 = jax.nn.sigmoid(pre_ref[0:_B, :])
    z = jnp.zeros((_B, _H), f32)
    h0, c0 = _gates(s0, z, 0)
    b1v = b1_ref[:, :]

    def substep(t, h0, c0, h1, c1, acc):
        g0 = jnp.dot(h0.astype(bf), Whh0_ref[:, :],
                     preferred_element_type=f32)
        hcat = jnp.concatenate([h0, h1], axis=1).astype(bf)
        g1 = jnp.dot(hcat, C1_ref[:, :], preferred_element_type=f32)
        s0 = jax.nn.sigmoid(g0 + pre_ref[pl.ds(t * _B, _B), :])
        s1 = jax.nn.sigmoid(g1 + b1v)
        h0n, c0n = _gates(s0, c0, 0)
        h1n, c1n = _gates(s1, c1, 0)
        hs_ref[pl.ds((t - 1) * _B, _B), :] = h1n
        return h0n, c0n, h1n, c1n, acc + h1n

    def step(i, carry):
        h0, c0, h1, c1, acc = carry
        h0, c0, h1, c1, acc = substep(2 * i + 1, h0, c0, h1, c1, acc)
        h0, c0, h1, c1, acc = substep(2 * i + 2, h0, c0, h1, c1, acc)
        return (h0, c0, h1, c1, acc)

    h0, c0, h1, c1, acc = jax.lax.fori_loop(
        0, (_T - 2) // 2, step, (h0, c0, z, z, z))

    h0, c0, h1, c1, acc = substep(_T - 1, h0, c0, h1, c1, acc)
    hcat = jnp.concatenate([h0, h1], axis=1).astype(bf)
    g1 = jnp.dot(hcat, C1_ref[:, :], preferred_element_type=f32)
    s = jax.nn.sigmoid(g1 + b1v)
    h1, c1 = _gates(s, c1, 0)
    hs_ref[(_T - 1) * _B:_T * _B, :] = h1
    acc = acc + h1

    mean_feat = acc * (1.0 / _T)
    mm1 = jnp.dot(mean_feat, Wl1T_ref[:, :],
                  preferred_element_type=f32) + bl1_ref[:, :]
    Wr1T = Wr1T_ref[:, :]

    def chunk(c, sumf1):
        rc = jnp.dot(hs_ref[pl.ds(c * _CT * _B, _CT * _B), :], Wr1T,
                     preferred_element_type=f32)
        f1c = _elu(rc.reshape(_CT, _B, _H1) + mm1[None, :, :])
        return sumf1 + jnp.sum(f1c, axis=0)

    sumf1 = jax.lax.fori_loop(0, _T // _CT, chunk, jnp.zeros((_B, _H1), f32))
    meanf1 = sumf1 * (1.0 / _T)
    f1_last = _elu(jnp.dot(hs_ref[(_T - 1) * _B:_T * _B, :], Wr1T,
                           preferred_element_type=f32) + mm1)

    f2 = (jnp.dot(meanf1, Wl2T_ref[:, :], preferred_element_type=f32)
          + bl2_ref[:, :]
          + jnp.dot(f1_last, Wr2T_ref[:, :], preferred_element_type=f32))

    zfc = jnp.maximum(
        jnp.dot(f2, W1T_ref[:, :], preferred_element_type=f32)
        + b1f_ref[:, :], 0.0)
    out_ref[:, :] = jnp.dot(zfc, W2bd_ref[:, :],
                            preferred_element_type=f32) + b2r_ref[:, :]


def kernel(x, W_ih0, W_hh0, b_ih0, b_hh0, W_ih1, W_hh1, b_ih1, b_hh1,
           Wl1, bl1, Wr1, Wl2, bl2, Wr2, fcW1, fcb1, fcW2, fcb2):
    f32 = jnp.float32
    bf = jnp.bfloat16
    xT = jnp.transpose(x, (1, 0, 2)).reshape(_T * _B, _IN)
    gscale1 = jnp.concatenate(
        [jnp.ones((2 * _H,), f32), jnp.full((_H,), 2.0, f32),
         jnp.ones((_H,), f32)])
    A0 = W_ih0.T * gscale1[None, :]
    b0 = ((b_ih0 + b_hh0) * gscale1).reshape(1, _G)
    b1 = ((b_ih1 + b_hh1) * gscale1).reshape(1, _G)
    Whh0s = (W_hh0.T * gscale1[None, :]).astype(bf)
    C1s = (jnp.concatenate([W_ih1.T, W_hh1.T], axis=0)
           * gscale1[None, :]).astype(bf)
    Wl1T = Wl1.T
    bl1r = bl1.reshape(1, _H1)
    Wr1T = Wr1.T
    Wl2T = Wl2.T
    bl2r = bl2.reshape(1, _OUTF)
    Wr2T = Wr2.T
    W1T = fcW1.reshape(_NFC * 64, _OUTF).T
    b1f = fcb1.reshape(1, _NFC * 64)
    eye = jnp.eye(_NFC, dtype=f32)
    W2bd = (fcW2[:, 0, :, None] * eye[:, None, :]).reshape(_NFC * 64, _NFC)
    b2r = fcb2.reshape(1, _NFC)

    out = pl.pallas_call(
        _fused_kernel,
        out_shape=jax.ShapeDtypeStruct((_B, _NFC), f32),
        scratch_shapes=[
            pltpu.VMEM((_T * _B, _G), f32),
            pltpu.VMEM((_T * _B, _H), f32),
        ],
    )(xT, A0, b0, b1, Whh0s, C1s, Wl1T, bl1r, Wr1T, Wl2T, bl2r, Wr2T,
      W1T, b1f, W2bd, b2r)
    return jnp.transpose(out)[:, :, None]


# speedup vs baseline: 443.2456x; 1.0639x over previous
import jax
import jax.numpy as jnp
from jax.experimental import pallas as pl
from jax.experimental.pallas import tpu as pltpu

_B, _T, _IN, _H = 8, 512, 8, 128
_H1, _OUTF, _OUTS, _NFC = 100, 128, 1, 8
_G = 4 * _H
_CT = 64


def _elu(v):
    return jnp.where(v > 0, v, jnp.exp(v) - 1.0)


def _gates(s, c, off):
    i = s[:, off:off + _H]
    f = s[:, off + _H:off + 2 * _H]
    g = 2.0 * s[:, off + 2 * _H:off + 3 * _H] - 1.0
    o = s[:, off + 3 * _H:off + 4 * _H]
    cn = f * c + i * g
    return o * jnp.tanh(cn), cn


def _fused_kernel(xT_ref, A0_ref, b0_ref, b1_ref, Whh0_ref, C1_ref,
                  Wl1T_ref, bl1_ref, Wr1T_ref, Wl2T_ref, bl2_ref, Wr2T_ref,
                  W1T_ref, b1f_ref, W2bd_ref, b2r_ref,
                  out_ref, pre_ref, hs_ref):
    f32 = jnp.float32
    bf = jnp.bfloat16
    pre_ref[:, :] = (
        jnp.dot(xT_ref[:, :], A0_ref[:, :], preferred_element_type=f32)
        + b0_ref[:, :])

    s0 = jax.nn.sigmoid(pre_ref[0:_B, :])
    z = jnp.zeros((_B, _H), f32)
    h0, c0 = _gates(s0, z, 0)
    b1v = b1_ref[:, :]

    def substep(t, h0, c0, h1, c1, acc, Whh0v, C1v):
        g0 = jnp.dot(h0.astype(bf), Whh0v, preferred_element_type=f32)
        hcat = jnp.concatenate([h0, h1], axis=1).astype(bf)
        g1 = jnp.dot(hcat, C1v, preferred_element_type=f32)
        s0 = jax.nn.sigmoid(g0 + pre_ref[pl.ds(t * _B, _B), :])
        s1 = jax.nn.sigmoid(g1 + b1v)
        h0n, c0n = _gates(s0, c0, 0)
        h1n, c1n = _gates(s1, c1, 0)
        hs_ref[pl.ds((t - 1) * _B, _B), :] = h1n
        return h0n, c0n, h1n, c1n, acc + h1n

    _U = 4

    def step(i, carry):
        h0, c0, h1, c1, acc = carry
        Whh0v = Whh0_ref[:, :]
        C1v = C1_ref[:, :]
        for u in range(_U):
            h0, c0, h1, c1, acc = substep(
                _U * i + 1 + u, h0, c0, h1, c1, acc, Whh0v, C1v)
        return (h0, c0, h1, c1, acc)

    n_trips = (_T - 1 - 3) // _U
    h0, c0, h1, c1, acc = jax.lax.fori_loop(
        0, n_trips, step, (h0, c0, z, z, z))

    Whh0v = Whh0_ref[:, :]
    C1v = C1_ref[:, :]
    for t in range(_T - 3, _T):
        h0, c0, h1, c1, acc = substep(t, h0, c0, h1, c1, acc, Whh0v, C1v)
    hcat = jnp.concatenate([h0, h1], axis=1).astype(bf)
    g1 = jnp.dot(hcat, C1_ref[:, :], preferred_element_type=f32)
    s = jax.nn.sigmoid(g1 + b1v)
    h1, c1 = _gates(s, c1, 0)
    hs_ref[(_T - 1) * _B:_T * _B, :] = h1
    acc = acc + h1

    mean_feat = acc * (1.0 / _T)
    mm1 = jnp.dot(mean_feat, Wl1T_ref[:, :],
                  preferred_element_type=f32) + bl1_ref[:, :]
    Wr1T = Wr1T_ref[:, :]

    def chunk(c, sumf1):
        rc = jnp.dot(hs_ref[pl.ds(c * _CT * _B, _CT * _B), :], Wr1T,
                     preferred_element_type=f32)
        f1c = _elu(rc.reshape(_CT, _B, _H1) + mm1[None, :, :])
        return sumf1 + jnp.sum(f1c, axis=0)

    sumf1 = jax.lax.fori_loop(0, _T // _CT, chunk, jnp.zeros((_B, _H1), f32))
    meanf1 = sumf1 * (1.0 / _T)
    f1_last = _elu(jnp.dot(hs_ref[(_T - 1) * _B:_T * _B, :], Wr1T,
                           preferred_element_type=f32) + mm1)

    f2 = (jnp.dot(meanf1, Wl2T_ref[:, :], preferred_element_type=f32)
          + bl2_ref[:, :]
          + jnp.dot(f1_last, Wr2T_ref[:, :], preferred_element_type=f32))

    zfc = jnp.maximum(
        jnp.dot(f2, W1T_ref[:, :], preferred_element_type=f32)
        + b1f_ref[:, :], 0.0)
    out_ref[:, :] = jnp.dot(zfc, W2bd_ref[:, :],
                            preferred_element_type=f32) + b2r_ref[:, :]


def kernel(x, W_ih0, W_hh0, b_ih0, b_hh0, W_ih1, W_hh1, b_ih1, b_hh1,
           Wl1, bl1, Wr1, Wl2, bl2, Wr2, fcW1, fcb1, fcW2, fcb2):
    f32 = jnp.float32
    bf = jnp.bfloat16
    xT = jnp.transpose(x, (1, 0, 2)).reshape(_T * _B, _IN)
    gscale1 = jnp.concatenate(
        [jnp.ones((2 * _H,), f32), jnp.full((_H,), 2.0, f32),
         jnp.ones((_H,), f32)])
    A0 = W_ih0.T * gscale1[None, :]
    b0 = ((b_ih0 + b_hh0) * gscale1).reshape(1, _G)
    b1 = ((b_ih1 + b_hh1) * gscale1).reshape(1, _G)
    Whh0s = (W_hh0.T * gscale1[None, :]).astype(bf)
    C1s = (jnp.concatenate([W_ih1.T, W_hh1.T], axis=0)
           * gscale1[None, :]).astype(bf)
    Wl1T = Wl1.T
    bl1r = bl1.reshape(1, _H1)
    Wr1T = Wr1.T
    Wl2T = Wl2.T
    bl2r = bl2.reshape(1, _OUTF)
    Wr2T = Wr2.T
    W1T = fcW1.reshape(_NFC * 64, _OUTF).T
    b1f = fcb1.reshape(1, _NFC * 64)
    eye = jnp.eye(_NFC, dtype=f32)
    W2bd = (fcW2[:, 0, :, None] * eye[:, None, :]).reshape(_NFC * 64, _NFC)
    b2r = fcb2.reshape(1, _NFC)

    out = pl.pallas_call(
        _fused_kernel,
        out_shape=jax.ShapeDtypeStruct((_B, _NFC), f32),
        scratch_shapes=[
            pltpu.VMEM((_T * _B, _G), f32),
            pltpu.VMEM((_T * _B, _H), f32),
        ],
    )(xT, A0, b0, b1, Whh0s, C1s, Wl1T, bl1r, Wr1T, Wl2T, bl2r, Wr2T,
      W1T, b1f, W2bd, b2r)
    return jnp.transpose(out)[:, :, None]
